# Initial kernel scaffold; baseline (speedup 1.0000x reference)
import jax
import jax.numpy as jnp
from jax.experimental import pallas as pl


def kernel(vecs, edge_index, adj_values, nnz, len_feat, neigh_weights, self_weights, offset, scale):
    raise NotImplementedError("write your pallas kernel here")


# speedup vs baseline: 3.6937x; 3.6937x over previous
import functools

import jax
import jax.numpy as jnp
from jax import lax
from jax.experimental import pallas as pl
from jax.experimental.pallas import tpu as pltpu
from jax.experimental.pallas import tpu_sc as plsc

N = 10000
D = 128
NC = 2
NS = 16
L = 16
NW = NC * NS

CH = 128
CHUNKS = 80
EPW = CHUNKS * CH
E_PAD = NW * EPW
N_PAD = 10240
RPT = N_PAD // NS


def _sc_segment_sum(vecs, src, dst, adj, zeros):
  mesh = plsc.VectorSubcoreMesh(
      core_axis_name="c", subcore_axis_name="s",
      num_cores=NC, num_subcores=NS)

  def body(vecs_h, src_h, dst_h, adj_h, zeros_h, out_h,
           src_v, dst_v, adj_v, rows_v, acc, sem):
    c = lax.axis_index("c")
    s = lax.axis_index("s")
    wid = s * NC + c

    for i in range(RPT // CH):
      pltpu.sync_copy(zeros_h, acc.at[pl.ds(s * RPT + i * CH, CH)])

    pltpu.sync_copy(src_h.at[wid], src_v)
    pltpu.sync_copy(dst_h.at[wid], dst_v)
    pltpu.sync_copy(adj_h.at[wid], adj_v)
    plsc.subcore_barrier()

    def chunk_body(ci, carry):
      pltpu.async_copy(vecs_h.at[src_v.at[ci]], rows_v, sem).wait()

      def row_body(rb, carry2):
        a16 = adj_v[ci, pl.ds(rb * L, L)]
        for i in range(L):
          a = a16[i]
          r = rb * L + i
          for g in range(D // L):
            sl = pl.ds(g * L, L)
            rows_v[r, sl] = rows_v[r, sl] * a
        return carry2

      lax.fori_loop(0, CH // L, row_body, 0)

      pltpu.sync_copy(rows_v, acc.at[dst_v.at[ci]], add=True)
      return carry

    lax.fori_loop(0, CHUNKS, chunk_body, 0)
    plsc.subcore_barrier()

    for i in range(RPT // CH):
      off = s * RPT + i * CH
      pltpu.sync_copy(acc.at[pl.ds(off, CH)], out_h.at[c, pl.ds(off, CH)])

  fn = pl.kernel(
      body,
      out_type=jax.ShapeDtypeStruct((NC, N_PAD, D), jnp.float32),
      mesh=mesh,
      scratch_types=[
          pltpu.VMEM((CHUNKS, CH), jnp.int32),
          pltpu.VMEM((CHUNKS, CH), jnp.int32),
          pltpu.VMEM((CHUNKS, CH), jnp.float32),
          pltpu.VMEM((CH, D), jnp.float32),
          pltpu.VMEM_SHARED((N_PAD, D), jnp.float32),
          pltpu.SemaphoreType.DMA,
      ],
  )
  return fn(vecs, src, dst, adj, zeros)


def _tc_dense(vecs, p0, p1, neigh_weights, self_weights, offset, scale):
  BR = 1000

  def body(v_ref, p0_ref, p1_ref, wn_ref, ws_ref, off_ref, sc_ref, o_ref):
    v = v_ref[...]
    nm = p0_ref[...] + p1_ref[...]
    fs = jnp.dot(v, ws_ref[...], preferred_element_type=jnp.float32)
    fn = jnp.dot(nm, wn_ref[...], preferred_element_type=jnp.float32)
    out = jnp.concatenate([fs, fn], axis=1)
    mean = jnp.mean(out, axis=1, keepdims=True)
    var = jnp.mean(jnp.square(out - mean), axis=1, keepdims=True)
    out = (out - mean) / jnp.sqrt(var + 1e-9) * sc_ref[...] + off_ref[...]
    o_ref[...] = jnp.maximum(out, 0.0)

  return pl.pallas_call(
      body,
      grid=(N // BR,),
      in_specs=[
          pl.BlockSpec((BR, D), lambda i: (i, 0)),
          pl.BlockSpec((BR, D), lambda i: (i, 0)),
          pl.BlockSpec((BR, D), lambda i: (i, 0)),
          pl.BlockSpec((D, D), lambda i: (0, 0)),
          pl.BlockSpec((D, D), lambda i: (0, 0)),
          pl.BlockSpec((1, 2 * D), lambda i: (0, 0)),
          pl.BlockSpec((1, 2 * D), lambda i: (0, 0)),
      ],
      out_specs=pl.BlockSpec((BR, 2 * D), lambda i: (i, 0)),
      out_shape=jax.ShapeDtypeStruct((N, 2 * D), jnp.float32),
  )(vecs, p0, p1, neigh_weights, self_weights, offset, scale)


def kernel(vecs, edge_index, adj_values, nnz, len_feat,
           neigh_weights, self_weights, offset, scale):
  del nnz, len_feat
  E = edge_index.shape[1]
  pad = E_PAD - E
  src = jnp.concatenate([edge_index[0], jnp.zeros((pad,), jnp.int32)])
  dst = jnp.concatenate([edge_index[1], jnp.zeros((pad,), jnp.int32)])
  adj = jnp.concatenate([adj_values, jnp.zeros((pad,), jnp.float32)])
  src = src.reshape(NW, CHUNKS, CH)
  dst = dst.reshape(NW, CHUNKS, CH)
  adj = adj.reshape(NW, CHUNKS, CH)
  zeros = jnp.zeros((CH, D), jnp.float32)

  partials = _sc_segment_sum(vecs, src, dst, adj, zeros)
  p0 = partials[0, :N]
  p1 = partials[1, :N]
  return _tc_dense(vecs, p0, p1, neigh_weights, self_weights, offset, scale)


# speedup vs baseline: 4.4029x; 1.1920x over previous
import functools

import jax
import jax.numpy as jnp
from jax import lax
from jax.experimental import pallas as pl
from jax.experimental.pallas import tpu as pltpu
from jax.experimental.pallas import tpu_sc as plsc

N = 10000
D = 128
NC = 2
NS = 16
L = 16
NW = NC * NS

CH = 64
CHUNKS = 160
EPW = CHUNKS * CH
E_PAD = NW * EPW
N_PAD = 10240
RPT = N_PAD // NS
NSL = 4
GROUPS = CHUNKS // NSL


def _sc_segment_sum(vecs, pk, zeros):
  mesh = plsc.VectorSubcoreMesh(
      core_axis_name="c", subcore_axis_name="s",
      num_cores=NC, num_subcores=NS)

  def body(vecs_h, pk_h, zeros_h, out_h, acc, rows, pkv, gsem, ssem, psem):
    c = lax.axis_index("c")
    s = lax.axis_index("s")
    wid = s * NC + c

    def wrap(x):
      return jnp.where(x >= CHUNKS, x - CHUNKS, x)

    def fire_pk(ci, slot):
      pltpu.async_copy(pk_h.at[wid, ci], pkv[slot], psem[slot])

    def wait_pk(slot):
      pltpu.make_async_copy(pk_h.at[0, 0], pkv[slot], psem[slot]).wait()

    def fire_gather(slot):
      pltpu.async_copy(vecs_h.at[pkv[slot].at[0]], rows[slot], gsem[slot])

    def wait_gather(slot):
      pltpu.make_async_copy(
          vecs_h.at[pl.ds(0, CH)], rows[slot], gsem[slot]).wait()

    def fire_scatter(slot):
      pltpu.async_copy(rows[slot], acc.at[pkv[slot].at[1]], ssem[slot],
                       add=True)

    def wait_scatter(slot):
      pltpu.make_async_copy(
          zeros_h.at[pl.ds(0, CH)], rows[slot], ssem[slot]).wait()

    for i in range(RPT // 128):
      pltpu.sync_copy(zeros_h, acc.at[pl.ds(s * RPT + i * 128, 128)])
    plsc.subcore_barrier()

    for j in range(3):
      fire_pk(j, j)
    wait_pk(0)
    pltpu.sync_copy(zeros_h.at[pl.ds(0, CH)], rows[NSL - 1])
    pltpu.async_copy(rows[NSL - 1], acc.at[pkv[0].at[1]], ssem[NSL - 1],
                     add=True)
    fire_gather(0)
    wait_pk(1)
    fire_gather(1)

    def group_body(g, carry):
      base = g * NSL
      for k in range(NSL):
        ci = base + k
        b = k
        wait_gather(b)

        def row_body(rb, carry2, _b=b):
          a16 = pkv[_b][2, pl.ds(rb * L, L)]
          for i in range(L):
            a = lax.bitcast_convert_type(a16[i], jnp.float32)
            r = rb * L + i
            for gg in range(D // L):
              sl = pl.ds(gg * L, L)
              rows[_b][r, sl] = rows[_b][r, sl] * a
          return carry2

        lax.fori_loop(0, CH // L, row_body, 0)
        fire_scatter(b)
        b3 = (k + 3) % NSL
        wait_scatter(b3)
        fire_pk(wrap(ci + 3), b3)
        b2 = (k + 2) % NSL
        wait_pk(b2)
        fire_gather(b2)
      return carry

    lax.fori_loop(0, GROUPS, group_body, 0)

    wait_scatter((CHUNKS - 1) % NSL)
    for j in range(2):
      wait_gather(j % NSL)
    wait_pk(2 % NSL)
    plsc.subcore_barrier()

    for i in range(RPT // 128):
      off = s * RPT + i * 128
      pltpu.sync_copy(acc.at[pl.ds(off, 128)], out_h.at[c, pl.ds(off, 128)])

  fn = pl.kernel(
      body,
      out_type=jax.ShapeDtypeStruct((NC, N_PAD, D), jnp.float32),
      mesh=mesh,
      scratch_types=[
          pltpu.VMEM_SHARED((N_PAD, D), jnp.float32),
          [pltpu.VMEM((CH, D), jnp.float32) for _ in range(NSL)],
          [pltpu.VMEM((3, CH), jnp.int32) for _ in range(NSL)],
          [pltpu.SemaphoreType.DMA for _ in range(NSL)],
          [pltpu.SemaphoreType.DMA for _ in range(NSL)],
          [pltpu.SemaphoreType.DMA for _ in range(NSL)],
      ],
  )
  return fn(vecs, pk, zeros)


def _tc_dense(vecs, partials, neigh_weights, self_weights, offset, scale):
  BR = 1000

  def body(v_ref, p0_ref, p1_ref, wn_ref, ws_ref, off_ref, sc_ref, o_ref):
    v = v_ref[...]
    nm = p0_ref[0] + p1_ref[0]
    fs = jnp.dot(v, ws_ref[...], preferred_element_type=jnp.float32)
    fn = jnp.dot(nm, wn_ref[...], preferred_element_type=jnp.float32)
    out = jnp.concatenate([fs, fn], axis=1)
    mean = jnp.mean(out, axis=1, keepdims=True)
    var = jnp.mean(jnp.square(out - mean), axis=1, keepdims=True)
    out = (out - mean) / jnp.sqrt(var + 1e-9) * sc_ref[...] + off_ref[...]
    o_ref[...] = jnp.maximum(out, 0.0)

  return pl.pallas_call(
      body,
      grid=(N // BR,),
      in_specs=[
          pl.BlockSpec((BR, D), lambda i: (i, 0)),
          pl.BlockSpec((1, BR, D), lambda i: (0, i, 0)),
          pl.BlockSpec((1, BR, D), lambda i: (1, i, 0)),
          pl.BlockSpec((D, D), lambda i: (0, 0)),
          pl.BlockSpec((D, D), lambda i: (0, 0)),
          pl.BlockSpec((1, 2 * D), lambda i: (0, 0)),
          pl.BlockSpec((1, 2 * D), lambda i: (0, 0)),
      ],
      out_specs=pl.BlockSpec((BR, 2 * D), lambda i: (i, 0)),
      out_shape=jax.ShapeDtypeStruct((N, 2 * D), jnp.float32),
  )(vecs, partials, partials, neigh_weights, self_weights, offset, scale)


def kernel(vecs, edge_index, adj_values, nnz, len_feat,
           neigh_weights, self_weights, offset, scale):
  del nnz, len_feat
  E = edge_index.shape[1]
  pad = E_PAD - E
  src = jnp.concatenate([edge_index[0], jnp.zeros((pad,), jnp.int32)])
  dst = jnp.concatenate([edge_index[1], jnp.zeros((pad,), jnp.int32)])
  adj = jnp.concatenate([adj_values, jnp.zeros((pad,), jnp.float32)])
  adj_bits = lax.bitcast_convert_type(adj, jnp.int32)
  pk = jnp.stack([src.reshape(NW, CHUNKS, CH),
                  dst.reshape(NW, CHUNKS, CH),
                  adj_bits.reshape(NW, CHUNKS, CH)], axis=2)
  zeros = jnp.zeros((128, D), jnp.float32)

  partials = _sc_segment_sum(vecs, pk, zeros)
  return _tc_dense(vecs, partials, neigh_weights, self_weights, offset, scale)
